# Initial kernel scaffold; baseline (speedup 1.0000x reference)
import jax
import jax.numpy as jnp
from jax.experimental import pallas as pl


def kernel(x, row, col, edge_weight, p, beta):
    raise NotImplementedError("write your pallas kernel here")


# speedup vs baseline: 6.3272x; 6.3272x over previous
import functools

import jax
import jax.numpy as jnp
from jax import lax
from jax.experimental import pallas as pl
from jax.experimental.pallas import tpu as pltpu
from jax.experimental.pallas import tpu_sc as plsc

_N = 10000
_NP = 10240
_E = 320000
_D = 128
_NW = 32
_CHUNK = 128
_NCHUNK = 80
_EP = _NW * _NCHUNK * _CHUNK
_STRIPE = _NP // 16

_sc_mesh = plsc.VectorSubcoreMesh(core_axis_name="c", subcore_axis_name="s")


@functools.partial(
    pl.kernel,
    mesh=_sc_mesh,
    out_type=(
        jax.ShapeDtypeStruct((_NP,), jnp.float32),
        jax.ShapeDtypeStruct((_NP,), jnp.float32),
    ),
    scratch_types=[
        pltpu.VMEM((_NCHUNK, _CHUNK), jnp.int32),
        pltpu.VMEM((_NCHUNK, _CHUNK), jnp.float32),
        pltpu.VMEM((_STRIPE,), jnp.float32),
        pltpu.VMEM_SHARED((_NP,), jnp.float32),
    ],
)
def _segment_sum_sc(col_hbm, ew_hbm, out0, out1, idx_v, val_v, zbuf, acc):
    c = lax.axis_index("c")
    s = lax.axis_index("s")
    wid = c * 16 + s

    pltpu.sync_copy(col_hbm.at[wid], idx_v)
    pltpu.sync_copy(ew_hbm.at[wid], val_v)

    for j in range(_STRIPE // 16):
        zbuf[pl.ds(j * 16, 16)] = jnp.zeros((16,), jnp.float32)
    pltpu.sync_copy(zbuf, acc.at[pl.ds(s * _STRIPE, _STRIPE)])
    plsc.subcore_barrier()

    def body(j, carry):
        pltpu.sync_copy(val_v.at[j], acc.at[idx_v.at[j]], add=True)
        return carry

    lax.fori_loop(0, _NCHUNK, body, 0)
    plsc.subcore_barrier()

    @pl.when(jnp.logical_and(s == 0, c == 0))
    def _():
        pltpu.sync_copy(acc, out0)

    @pl.when(jnp.logical_and(s == 0, c == 1))
    def _():
        pltpu.sync_copy(acc, out1)


_R = 2000


def _pan_tc_body(x_ref, p_ref, p0_ref, p1_ref, beta_ref, out_ref, score_ref):
    xb = x_ref[...]
    s1 = jnp.sum(xb * p_ref[...], axis=1, keepdims=True)
    s2 = p0_ref[...] + p1_ref[...]
    z = beta_ref[0] * s1 + beta_ref[1] * s2
    sc = 1.0 / (1.0 + jnp.exp(-z))
    out_ref[...] = xb * sc
    score_ref[...] = sc


def kernel(x, row, col, edge_weight, p, beta):
    del row
    pad = _EP - _E
    colp = jnp.concatenate([col, jnp.zeros((pad,), jnp.int32)])
    ewp = jnp.concatenate([edge_weight, jnp.zeros((pad,), jnp.float32)])
    colp = colp.reshape(_NW, _NCHUNK, _CHUNK)
    ewp = ewp.reshape(_NW, _NCHUNK, _CHUNK)

    part0, part1 = _segment_sum_sc(colp, ewp)

    out, score = pl.pallas_call(
        _pan_tc_body,
        grid=(_N // _R,),
        in_specs=[
            pl.BlockSpec((_R, _D), lambda i: (i, 0)),
            pl.BlockSpec((1, _D), lambda i: (0, 0)),
            pl.BlockSpec((_R, 1), lambda i: (i, 0)),
            pl.BlockSpec((_R, 1), lambda i: (i, 0)),
            pl.BlockSpec(memory_space=pltpu.SMEM),
        ],
        out_specs=[
            pl.BlockSpec((_R, _D), lambda i: (i, 0)),
            pl.BlockSpec((_R, 1), lambda i: (i, 0)),
        ],
        out_shape=[
            jax.ShapeDtypeStruct((_N, _D), jnp.float32),
            jax.ShapeDtypeStruct((_N, 1), jnp.float32),
        ],
    )(x, p.reshape(1, _D), part0.reshape(_NP, 1), part1.reshape(_NP, 1), beta)

    return (out, score.reshape(_N))


# speedup vs baseline: 6.6598x; 1.0526x over previous
import functools

import jax
import jax.numpy as jnp
from jax import lax
from jax.experimental import pallas as pl
from jax.experimental.pallas import tpu as pltpu
from jax.experimental.pallas import tpu_sc as plsc

_N = 10000
_NP = 10240
_E = 320000
_D = 128
_NW = 32
_CHUNK = 125
_NCHUNK = 80
_DEPTH = 16
_STRIPE = _NP // 16

_sc_mesh = plsc.VectorSubcoreMesh(core_axis_name="c", subcore_axis_name="s")


@functools.partial(
    pl.kernel,
    mesh=_sc_mesh,
    out_type=(
        jax.ShapeDtypeStruct((_NP,), jnp.float32),
        jax.ShapeDtypeStruct((_NP,), jnp.float32),
    ),
    scratch_types=[
        pltpu.VMEM((_NCHUNK, _CHUNK), jnp.int32),
        pltpu.VMEM((_NCHUNK, _CHUNK), jnp.float32),
        pltpu.VMEM((_STRIPE,), jnp.float32),
        pltpu.VMEM_SHARED((_NP,), jnp.float32),
        pltpu.SemaphoreType.DMA,
        pltpu.SemaphoreType.DMA,
    ],
)
def _segment_sum_sc(col_hbm, ew_hbm, out0, out1, idx_v, val_v, zbuf, acc,
                    ld_sem, st_sem):
    c = lax.axis_index("c")
    s = lax.axis_index("s")
    wid = c * 16 + s

    pltpu.async_copy(col_hbm.at[wid], idx_v, ld_sem)
    pltpu.async_copy(ew_hbm.at[wid], val_v, ld_sem)

    for j in range(_STRIPE // 16):
        zbuf[pl.ds(j * 16, 16)] = jnp.zeros((16,), jnp.float32)
    pltpu.sync_copy(zbuf, acc.at[pl.ds(s * _STRIPE, _STRIPE)])
    pltpu.make_async_copy(col_hbm.at[wid], idx_v, ld_sem).wait()
    pltpu.make_async_copy(ew_hbm.at[wid], val_v, ld_sem).wait()
    plsc.subcore_barrier()

    def fire(j, carry):
        pltpu.async_copy(val_v.at[j], acc.at[idx_v.at[j]], st_sem, add=True)
        return carry

    def wait_fire(j, carry):
        pltpu.make_async_copy(
            val_v.at[j - _DEPTH], acc.at[idx_v.at[j - _DEPTH]], st_sem).wait()
        pltpu.async_copy(val_v.at[j], acc.at[idx_v.at[j]], st_sem, add=True)
        return carry

    def drain(j, carry):
        pltpu.make_async_copy(val_v.at[j], acc.at[idx_v.at[j]], st_sem).wait()
        return carry

    lax.fori_loop(0, _DEPTH, fire, 0)
    lax.fori_loop(_DEPTH, _NCHUNK, wait_fire, 0)
    lax.fori_loop(_NCHUNK - _DEPTH, _NCHUNK, drain, 0)
    plsc.subcore_barrier()

    @pl.when(jnp.logical_and(s == 0, c == 0))
    def _():
        pltpu.sync_copy(acc, out0)

    @pl.when(jnp.logical_and(s == 0, c == 1))
    def _():
        pltpu.sync_copy(acc, out1)


_R = 2000


def _pan_tc_body(x_ref, p_ref, p0_ref, p1_ref, beta_ref, out_ref, score_ref):
    xb = x_ref[...]
    s1 = jnp.sum(xb * p_ref[...], axis=1, keepdims=True)
    s2 = p0_ref[...] + p1_ref[...]
    z = beta_ref[0] * s1 + beta_ref[1] * s2
    sc = 1.0 / (1.0 + jnp.exp(-z))
    out_ref[...] = xb * sc
    score_ref[...] = sc


def kernel(x, row, col, edge_weight, p, beta):
    del row
    colp = col.reshape(_NW, _NCHUNK, _CHUNK)
    ewp = edge_weight.reshape(_NW, _NCHUNK, _CHUNK)

    part0, part1 = _segment_sum_sc(colp, ewp)

    out, score = pl.pallas_call(
        _pan_tc_body,
        grid=(_N // _R,),
        in_specs=[
            pl.BlockSpec((_R, _D), lambda i: (i, 0)),
            pl.BlockSpec((1, _D), lambda i: (0, 0)),
            pl.BlockSpec((_R, 1), lambda i: (i, 0)),
            pl.BlockSpec((_R, 1), lambda i: (i, 0)),
            pl.BlockSpec(memory_space=pltpu.SMEM),
        ],
        out_specs=[
            pl.BlockSpec((_R, _D), lambda i: (i, 0)),
            pl.BlockSpec((_R, 1), lambda i: (i, 0)),
        ],
        out_shape=[
            jax.ShapeDtypeStruct((_N, _D), jnp.float32),
            jax.ShapeDtypeStruct((_N, 1), jnp.float32),
        ],
    )(x, p.reshape(1, _D), part0.reshape(_NP, 1), part1.reshape(_NP, 1), beta)

    return (out, score.reshape(_N))


# speedup vs baseline: 10.9292x; 1.6411x over previous
import functools

import jax
import jax.numpy as jnp
from jax import lax
from jax.experimental import pallas as pl
from jax.experimental.pallas import tpu as pltpu
from jax.experimental.pallas import tpu_sc as plsc

_N = 10000
_NP = 10240
_E = 320000
_D = 128
_NW = 32
_PER_W = _E // _NW
_CHUNK = 128
_NCHUNK = _PER_W // _CHUNK
_TAIL = _PER_W - _NCHUNK * _CHUNK
_DEPTH = 16
_STRIPE = _NP // 16

_sc_mesh = plsc.VectorSubcoreMesh(core_axis_name="c", subcore_axis_name="s")


@functools.partial(
    pl.kernel,
    mesh=_sc_mesh,
    out_type=(
        jax.ShapeDtypeStruct((_NP,), jnp.float32),
        jax.ShapeDtypeStruct((_NP,), jnp.float32),
    ),
    scratch_types=[
        pltpu.VMEM((_NCHUNK, _CHUNK), jnp.int32),
        pltpu.VMEM((1, _TAIL), jnp.int32),
        pltpu.VMEM((_PER_W,), jnp.float32),
        pltpu.VMEM((_STRIPE,), jnp.float32),
        pltpu.VMEM_SHARED((_NP,), jnp.float32),
        pltpu.SemaphoreType.DMA,
        pltpu.SemaphoreType.DMA,
    ],
)
def _segment_sum_sc(col_hbm, ew_hbm, out0, out1, idx_v, idx_t, val_v, zbuf,
                    acc, ld_sem, st_sem):
    c = lax.axis_index("c")
    s = lax.axis_index("s")
    wid = c * 16 + s
    base = wid * _PER_W

    pltpu.async_copy(ew_hbm.at[pl.ds(base, _PER_W)], val_v, ld_sem)
    pltpu.async_copy(col_hbm.at[pl.ds(base + _NCHUNK * _CHUNK, _TAIL)],
                     idx_t.at[0], ld_sem)

    def ld(j, carry):
        pltpu.async_copy(col_hbm.at[pl.ds(base + j * _CHUNK, _CHUNK)],
                         idx_v.at[j], ld_sem)
        return carry

    lax.fori_loop(0, _NCHUNK, ld, 0)

    for j in range(_STRIPE // 16):
        zbuf[pl.ds(j * 16, 16)] = jnp.zeros((16,), jnp.float32)
    pltpu.sync_copy(zbuf, acc.at[pl.ds(s * _STRIPE, _STRIPE)])

    pltpu.make_async_copy(ew_hbm.at[pl.ds(base, _PER_W)], val_v, ld_sem).wait()
    pltpu.make_async_copy(col_hbm.at[pl.ds(base, _TAIL)], idx_t.at[0],
                          ld_sem).wait()

    def ld_drain(j, carry):
        pltpu.make_async_copy(col_hbm.at[pl.ds(base, _CHUNK)], idx_v.at[j],
                              ld_sem).wait()
        return carry

    lax.fori_loop(0, _NCHUNK, ld_drain, 0)
    plsc.subcore_barrier()

    def fire(j, carry):
        pltpu.async_copy(val_v.at[pl.ds(j * _CHUNK, _CHUNK)],
                         acc.at[idx_v.at[j]], st_sem, add=True)
        return carry

    def wait_fire(j, carry):
        k = j - _DEPTH
        pltpu.make_async_copy(val_v.at[pl.ds(k * _CHUNK, _CHUNK)],
                              acc.at[idx_v.at[k]], st_sem).wait()
        pltpu.async_copy(val_v.at[pl.ds(j * _CHUNK, _CHUNK)],
                         acc.at[idx_v.at[j]], st_sem, add=True)
        return carry

    def drain(j, carry):
        pltpu.make_async_copy(val_v.at[pl.ds(j * _CHUNK, _CHUNK)],
                              acc.at[idx_v.at[j]], st_sem).wait()
        return carry

    lax.fori_loop(0, _DEPTH, fire, 0)
    lax.fori_loop(_DEPTH, _NCHUNK, wait_fire, 0)
    pltpu.async_copy(val_v.at[pl.ds(_NCHUNK * _CHUNK, _TAIL)],
                     acc.at[idx_t.at[0]], st_sem, add=True)
    lax.fori_loop(_NCHUNK - _DEPTH, _NCHUNK, drain, 0)
    pltpu.make_async_copy(val_v.at[pl.ds(_NCHUNK * _CHUNK, _TAIL)],
                          acc.at[idx_t.at[0]], st_sem).wait()
    plsc.subcore_barrier()

    @pl.when(jnp.logical_and(s == 0, c == 0))
    def _():
        pltpu.sync_copy(acc, out0)

    @pl.when(jnp.logical_and(s == 0, c == 1))
    def _():
        pltpu.sync_copy(acc, out1)


_R = 2048


def _pan_tc_body(x_ref, p_ref, p0_ref, p1_ref, beta_ref, out_ref, score_ref):
    xb = x_ref[...]
    s1 = jnp.sum(xb * p_ref[...], axis=1, keepdims=True)
    pp = p0_ref[...] + p1_ref[...]
    ppt = jnp.transpose(pp)
    s2 = jnp.concatenate([ppt[:, r:r + 1] for r in range(_R // 128)], axis=0)
    z = beta_ref[0] * s1 + beta_ref[1] * s2
    sc = 1.0 / (1.0 + jnp.exp(-z))
    out_ref[...] = xb * sc
    sc_cols = jnp.concatenate(
        [sc[128 * r:128 * (r + 1), :] for r in range(_R // 128)], axis=1)
    score_ref[...] = jnp.transpose(sc_cols)


def kernel(x, row, col, edge_weight, p, beta):
    del row
    part0, part1 = _segment_sum_sc(col, edge_weight)

    nblk = _R // 128
    out, score = pl.pallas_call(
        _pan_tc_body,
        grid=(pl.cdiv(_N, _R),),
        in_specs=[
            pl.BlockSpec((_R, _D), lambda i: (i, 0)),
            pl.BlockSpec((1, _D), lambda i: (0, 0)),
            pl.BlockSpec((nblk, 128), lambda i: (i, 0)),
            pl.BlockSpec((nblk, 128), lambda i: (i, 0)),
            pl.BlockSpec(memory_space=pltpu.SMEM),
        ],
        out_specs=[
            pl.BlockSpec((_R, _D), lambda i: (i, 0)),
            pl.BlockSpec((nblk, 128), lambda i: (i, 0)),
        ],
        out_shape=[
            jax.ShapeDtypeStruct((_N, _D), jnp.float32),
            jax.ShapeDtypeStruct((_NP // 128, 128), jnp.float32),
        ],
    )(x, p.reshape(1, _D), part0.reshape(_NP // 128, 128),
      part1.reshape(_NP // 128, 128), beta)

    return (out, score.reshape(_NP)[:_N])
